# baseline (speedup 1.0000x reference)
import functools

import jax
import jax.numpy as jnp
from jax import lax
from jax.experimental import pallas as pl
from jax.experimental.pallas import tpu as pltpu
from jax.experimental.pallas import tpu_sc as plsc

_B = 16384
_D = 64
_INFO = plsc.get_sparse_core_info()
_NC = _INFO.num_cores
_NS = _INFO.num_subcores
_L = _INFO.num_lanes
_NW = _NC * _NS
_BPW = _B // _NW
_CH = 128
_NCH = _BPW // _CH
_G = 16
_NG = _BPW // _G


def _distmult_body(h_idx_hbm, t_idx_hbm, r_idx_hbm, node_hbm, rel_hbm,
                   out_hbm, hi_v, ti_v, ri_v, h_rows, t_rows, r_rows,
                   tmp, scores_v, sem):
    wid = lax.axis_index("s") * _NC + lax.axis_index("c")
    base = wid * _BPW

    pltpu.sync_copy(h_idx_hbm.at[pl.ds(base, _BPW)], hi_v)
    pltpu.sync_copy(t_idx_hbm.at[pl.ds(base, _BPW)], ti_v)
    pltpu.sync_copy(r_idx_hbm.at[pl.ds(base, _BPW)], ri_v)

    copies = []
    for c in range(_NCH):
        s = pl.ds(c * _CH, _CH)
        copies.append(pltpu.async_copy(node_hbm.at[hi_v.at[s]], h_rows.at[s], sem))
        copies.append(pltpu.async_copy(node_hbm.at[ti_v.at[s]], t_rows.at[s], sem))
        copies.append(pltpu.async_copy(rel_hbm.at[ri_v.at[s]], r_rows.at[s], sem))
    for cp in copies:
        cp.wait()

    lane = lax.iota(jnp.int32, _L)

    def group(g, carry):
        rb = g * _G
        acc = jnp.zeros((_L,), jnp.float32)
        for j in range(_G):
            row = rb + j
            p = None
            for k in range(_D // _L):
                sl = pl.ds(k * _L, _L)
                prod = h_rows[row, sl] * r_rows[row, sl] * t_rows[row, sl]
                p = prod if p is None else p + prod
            acc = jnp.where(lane == j, jnp.sum(p), acc)
        scores_v[pl.ds(rb, _G)] = acc
        return carry

    lax.fori_loop(0, _NG, group, 0)
    pltpu.sync_copy(scores_v, out_hbm.at[pl.ds(base, _BPW)])


@functools.partial(
    pl.kernel,
    out_type=jax.ShapeDtypeStruct((_B,), jnp.float32),
    mesh=plsc.VectorSubcoreMesh(core_axis_name="c", subcore_axis_name="s"),
    compiler_params=pltpu.CompilerParams(needs_layout_passes=False,
                                         use_tc_tiling_on_sc=False),
    scratch_types=[
        pltpu.VMEM((_BPW,), jnp.int32),
        pltpu.VMEM((_BPW,), jnp.int32),
        pltpu.VMEM((_BPW,), jnp.int32),
        pltpu.VMEM((_BPW, _D), jnp.float32),
        pltpu.VMEM((_BPW, _D), jnp.float32),
        pltpu.VMEM((_BPW, _D), jnp.float32),
        pltpu.VMEM((_G * (_L + 1),), jnp.float32),
        pltpu.VMEM((_BPW,), jnp.float32),
        pltpu.SemaphoreType.DMA,
    ],
)
def _distmult_sc(h_idx, t_idx, r_idx, node_emb, rel_emb, out,
                 hi_v, ti_v, ri_v, h_rows, t_rows, r_rows, tmp, scores_v,
                 sem):
    _distmult_body(h_idx, t_idx, r_idx, node_emb, rel_emb, out,
                   hi_v, ti_v, ri_v, h_rows, t_rows, r_rows, tmp,
                   scores_v, sem)


def kernel(head_indices, tail_indices, relation_indices, node_embedding,
           relation_embedding):
    return _distmult_sc(head_indices.astype(jnp.int32),
                        tail_indices.astype(jnp.int32),
                        relation_indices.astype(jnp.int32),
                        node_embedding, relation_embedding)
